# baseline (speedup 1.0000x reference)
import functools

import jax
import jax.numpy as jnp
from jax import lax
from jax.experimental import pallas as pl
from jax.experimental.pallas import tpu as pltpu
from jax.experimental.pallas import tpu_sc as plsc

NC = 2
NS = 16
L = 16
NW = NC * NS
BATCH = 16384
FACTOR = 64
NROW = 1000000
NCHUNK = FACTOR // L
G = 6
GW = G * 128
NG = (NROW // 128 + G) // G
EXTENT = ((NROW + 127) // 128) * 128
MAXSTART = EXTENT - GW
NPAD = NW * L
NBUF = BATCH + NPAD
BPW = BATCH // NW
RHALF = BPW // 2
LISTCAP = BATCH + L
ISTREAM = 4096

_mesh = plsc.VectorSubcoreMesh(core_axis_name="c", subcore_axis_name="s")


@functools.partial(
    pl.kernel,
    mesh=_mesh,
    compiler_params=pltpu.CompilerParams(needs_layout_passes=False),
    out_type=(
        jax.ShapeDtypeStruct((NBUF, 128), jnp.float32),
        jax.ShapeDtypeStruct((NBUF, 128), jnp.float32),
    ),
    scratch_types=[
        pltpu.VMEM((ISTREAM,), jnp.int32),
        pltpu.VMEM((LISTCAP,), jnp.int32),
        pltpu.VMEM((FACTOR, GW), jnp.float32),
        pltpu.VMEM((FACTOR, GW), jnp.float32),
        pltpu.VMEM((L,), jnp.int32),
        pltpu.VMEM((L, 128), jnp.float32),
        pltpu.SemaphoreType.DMA,
        pltpu.SemaphoreType.DMA,
        pltpu.SemaphoreType.DMA,
    ],
)
def _gmf_scan(user_hbm, item_hbm, ut_t, it_t, u_buf, i_buf,
              idxbuf, list_e, col0, col1, ment, stage,
              sem0, sem1, semf):
    wid = lax.axis_index("s") * NC + lax.axis_index("c")
    g0 = (wid * NG) >> 5
    g1 = ((wid + 1) * NG) >> 5
    ng = g1 - g0
    npairs = (ng + 1) >> 1
    iota = lax.iota(jnp.int32, L)
    fidx = [iota + c * L for c in range(NCHUNK)]
    pad_ids = BATCH + wid * L + iota
    cols = [col0, col1]
    sems = [sem0, sem1]

    def run_table(tab_t, idx_hbm, out_buf):
        def gidx(gp, h):
            return g0 + jnp.minimum(2 * gp + h, ng - 1)

        def fetch(g, s):
            start = jnp.minimum(g * GW, MAXSTART)
            pltpu.async_copy(tab_t.at[:, pl.ds(start, GW)], cols[s], sems[s])

        fetch(gidx(0, 0), 0)
        fetch(gidx(0, 1), 1)

        def compact_round(r, n0):
            pltpu.sync_copy(
                idx_hbm.at[pl.ds(r * ISTREAM, ISTREAM)], idxbuf)

            def compact(ch, n):
                v = idxbuf[pl.ds(ch * L, L)]
                grp = v // GW
                m = (grp >= g0) & (grp < g1)
                lane = v - grp * GW
                entry = ((grp - g0) << 24) | (lane << 14) | (
                    r * ISTREAM + ch * L + iota)
                plsc.store_compressed(list_e.at[pl.ds(n, L)], entry, mask=m)
                return n + plsc.all_reduce_population_count(m)[0]

            return lax.fori_loop(0, ISTREAM // L, compact, n0)

        n = 0
        for r in range(BATCH // ISTREAM):
            n = compact_round(r, n)
        nch = (n + L - 1) >> 4

        def process(g, s, carry):
            pltpu.make_async_copy(
                tab_t.at[:, pl.ds(0, GW)], cols[s], sems[s]).wait()
            gstart = jnp.minimum(g * GW, MAXSTART)
            delta = g * GW - gstart

            def rescan(ch, c2):
                ev = list_e[pl.ds(ch * L, L)]
                valid = (ch * L + iota) < n
                m2 = ((ev >> 24) == (g - g0)) & valid
                plsc.store_compressed(ment.at[pl.ds(0, L)], ev, mask=m2)
                k = plsc.all_reduce_population_count(m2)[0]

                def match(j, c3):
                    cnt, ids_v = c3
                    jb = jnp.broadcast_to(j, (L,))
                    eb = plsc.load_gather(ment, [jb])
                    laneb = ((eb >> 14) & 1023) + delta
                    idb = eb & 16383
                    r = cnt & (L - 1)
                    for c in range(NCHUNK):
                        v = plsc.load_gather(cols[s], [fidx[c], laneb])
                        stage[r, pl.ds(c * L, L)] = v
                    ids_v = jnp.where(iota == r, idb, ids_v)
                    cnt = cnt + 1
                    flush = (cnt & (L - 1)) == 0

                    @pl.when(flush)
                    def _():
                        pltpu.async_copy(stage, out_buf.at[ids_v], semf).wait()

                    ids_v = jnp.where(jnp.broadcast_to(flush, (L,)),
                                      pad_ids, ids_v)
                    return cnt, ids_v

                return lax.fori_loop(0, k, match, c2)

            return lax.fori_loop(0, nch, rescan, carry)

        def pair(gp, carry):
            for h in (0, 1):
                g = gidx(gp, h)
                carry = process(g, h, carry)
                fetch(gidx(gp + 1, h), h)
            return carry

        carry = lax.fori_loop(0, npairs, pair, (0, pad_ids))
        _, ids_v = carry
        pltpu.async_copy(stage, out_buf.at[ids_v], semf).wait()
        for s in (0, 1):
            pltpu.make_async_copy(
                tab_t.at[:, pl.ds(0, GW)], cols[s], sems[s]).wait()

    run_table(ut_t, user_hbm, u_buf)
    run_table(it_t, item_hbm, i_buf)


@functools.partial(
    pl.kernel,
    mesh=_mesh,
    compiler_params=pltpu.CompilerParams(
        needs_layout_passes=False, use_tc_tiling_on_sc=False),
    out_type=jax.ShapeDtypeStruct((BATCH,), jnp.float32),
    scratch_types=[
        pltpu.VMEM((BPW, FACTOR), jnp.float32),
        pltpu.VMEM((BPW, FACTOR), jnp.float32),
        pltpu.VMEM((FACTOR,), jnp.float32),
        pltpu.VMEM((L,), jnp.float32),
        pltpu.VMEM((BPW,), jnp.float32),
        pltpu.SemaphoreType.DMA,
        pltpu.SemaphoreType.DMA,
    ],
)
def _gmf_compute(u_hbm, i_hbm, w_hbm, b_hbm, out_hbm,
                 u_rows, i_rows, w_v, b_v, out_v, sem_u, sem_i):
    wid = lax.axis_index("s") * NC + lax.axis_index("c")
    base = wid * BPW
    pltpu.sync_copy(w_hbm, w_v)
    pltpu.sync_copy(b_hbm, b_v)
    w = [w_v[pl.ds(c * L, L)] for c in range(NCHUNK)]
    bv = b_v[...]
    iota = lax.iota(jnp.int32, L)

    cu = pltpu.async_copy(
        u_hbm.at[pl.ds(base, BPW), pl.ds(0, FACTOR)], u_rows, sem_u)
    ci = pltpu.async_copy(
        i_hbm.at[pl.ds(base, BPW), pl.ds(0, FACTOR)], i_rows, sem_i)
    cu.wait()
    ci.wait()

    def group(gi, carry):
        rowbase = gi * L
        logit = jnp.zeros((L,), jnp.float32)
        for e in range(L):
            row = rowbase + e
            acc = None
            for c in range(NCHUNK):
                u = u_rows[row, pl.ds(c * L, L)]
                iv = i_rows[row, pl.ds(c * L, L)]
                p = u * iv * w[c]
                acc = p if acc is None else acc + p
            logit = jnp.where(iota == e, jnp.sum(acc), logit)
        logit = logit + bv
        out_v[pl.ds(rowbase, L)] = 1.0 / (1.0 + jnp.exp(-logit))
        return carry

    lax.fori_loop(0, BPW // L, group, 0)

    pltpu.sync_copy(out_v, out_hbm.at[pl.ds(base, BPW)])


def kernel(user, item, user_table, item_table, W, b):
    u_buf, i_buf = _gmf_scan(user, item, user_table.T, item_table.T)
    w64 = W.reshape((FACTOR,))
    b16 = jnp.broadcast_to(b, (L,))
    out = _gmf_compute(u_buf, i_buf, w64, b16)
    return out.reshape((BATCH, 1))


# speedup vs baseline: 1.2603x; 1.2603x over previous
import functools

import jax
import jax.numpy as jnp
from jax import lax
from jax.experimental import pallas as pl
from jax.experimental.pallas import tpu as pltpu
from jax.experimental.pallas import tpu_sc as plsc

NC = 2
NS = 16
L = 16
NW = NC * NS
BATCH = 16384
FACTOR = 64
NROW = 1000000
NCHUNK = FACTOR // L
G = 4
GW = G * 128
NG = (NROW // 128 + G) // G
EXTENT = ((NROW + 127) // 128) * 128
MAXSTART = EXTENT - GW
NPAD = NW * L
NBUF = BATCH + NPAD
BPW = BATCH // NW
RHALF = BPW // 2
LISTCAP = BATCH + L

_mesh = plsc.VectorSubcoreMesh(core_axis_name="c", subcore_axis_name="s")


@functools.partial(
    pl.kernel,
    mesh=_mesh,
    compiler_params=pltpu.CompilerParams(needs_layout_passes=False),
    out_type=(
        jax.ShapeDtypeStruct((NBUF, 128), jnp.float32),
        jax.ShapeDtypeStruct((NBUF, 128), jnp.float32),
    ),
    scratch_types=[
        pltpu.VMEM((BATCH,), jnp.int32),
        pltpu.VMEM((LISTCAP,), jnp.int32),
        pltpu.VMEM((LISTCAP,), jnp.int32),
        pltpu.VMEM((FACTOR, GW), jnp.float32),
        pltpu.VMEM((FACTOR, GW), jnp.float32),
        pltpu.VMEM((L,), jnp.int32),
        pltpu.VMEM((L,), jnp.int32),
        pltpu.VMEM((L, 128), jnp.float32),
        pltpu.SemaphoreType.DMA,
        pltpu.SemaphoreType.DMA,
        pltpu.SemaphoreType.DMA,
    ],
)
def _gmf_scan(user_hbm, item_hbm, ut_t, it_t, u_buf, i_buf,
              idxbuf, list_u, list_id, col0, col1, mlane, mid, stage,
              sem0, sem1, semf):
    wid = lax.axis_index("s") * NC + lax.axis_index("c")
    g0 = (wid * NG) >> 5
    g1 = ((wid + 1) * NG) >> 5
    ng = g1 - g0
    npairs = (ng + 1) >> 1
    iota = lax.iota(jnp.int32, L)
    fidx = [iota + c * L for c in range(NCHUNK)]
    pad_ids = BATCH + wid * L + iota
    cols = [col0, col1]
    sems = [sem0, sem1]

    def run_table(tab_t, idx_hbm, out_buf):
        def gidx(gp, h):
            return g0 + jnp.minimum(2 * gp + h, ng - 1)

        def fetch(g, s):
            start = jnp.minimum(g * GW, MAXSTART)
            pltpu.async_copy(tab_t.at[:, pl.ds(start, GW)], cols[s], sems[s])

        fetch(gidx(0, 0), 0)
        fetch(gidx(0, 1), 1)
        pltpu.sync_copy(idx_hbm, idxbuf)

        def compact(ch, n):
            v = idxbuf[pl.ds(ch * L, L)]
            grp = v >> 9
            m = (grp >= g0) & (grp < g1)
            ids = ch * L + iota
            plsc.store_compressed(list_u.at[pl.ds(n, L)], v, mask=m)
            plsc.store_compressed(list_id.at[pl.ds(n, L)], ids, mask=m)
            return n + plsc.all_reduce_population_count(m)[0]

        n = lax.fori_loop(0, BATCH // L, compact, 0)
        nch = (n + L - 1) >> 4

        def process(g, s, carry):
            pltpu.make_async_copy(
                tab_t.at[:, pl.ds(0, GW)], cols[s], sems[s]).wait()
            gstart = jnp.minimum(g * GW, MAXSTART)

            def rescan(ch, c2):
                lv = list_u[pl.ds(ch * L, L)]
                lid = list_id[pl.ds(ch * L, L)]
                valid = (ch * L + iota) < n
                m2 = ((lv >> 9) == g) & valid
                lwin = lv - gstart
                plsc.store_compressed(mlane.at[pl.ds(0, L)], lwin, mask=m2)
                plsc.store_compressed(mid.at[pl.ds(0, L)], lid, mask=m2)
                k = plsc.all_reduce_population_count(m2)[0]

                def match(j, c3):
                    cnt, ids_v = c3
                    jb = jnp.broadcast_to(j, (L,))
                    laneb = plsc.load_gather(mlane, [jb])
                    idb = plsc.load_gather(mid, [jb])
                    r = cnt & (L - 1)
                    for c in range(NCHUNK):
                        v = plsc.load_gather(cols[s], [fidx[c], laneb])
                        stage[r, pl.ds(c * L, L)] = v
                    ids_v = jnp.where(iota == r, idb, ids_v)
                    cnt = cnt + 1
                    flush = (cnt & (L - 1)) == 0

                    @pl.when(flush)
                    def _():
                        pltpu.async_copy(stage, out_buf.at[ids_v], semf).wait()

                    ids_v = jnp.where(jnp.broadcast_to(flush, (L,)),
                                      pad_ids, ids_v)
                    return cnt, ids_v

                return lax.fori_loop(0, k, match, c2)

            return lax.fori_loop(0, nch, rescan, carry)

        def pair(gp, carry):
            for h in (0, 1):
                g = gidx(gp, h)
                carry = process(g, h, carry)
                fetch(gidx(gp + 1, h), h)
            return carry

        carry = lax.fori_loop(0, npairs, pair, (0, pad_ids))
        _, ids_v = carry
        pltpu.async_copy(stage, out_buf.at[ids_v], semf).wait()
        for s in (0, 1):
            pltpu.make_async_copy(
                tab_t.at[:, pl.ds(0, GW)], cols[s], sems[s]).wait()

    run_table(ut_t, user_hbm, u_buf)
    run_table(it_t, item_hbm, i_buf)


@functools.partial(
    pl.kernel,
    mesh=_mesh,
    compiler_params=pltpu.CompilerParams(
        needs_layout_passes=False, use_tc_tiling_on_sc=False),
    out_type=jax.ShapeDtypeStruct((BATCH,), jnp.float32),
    scratch_types=[
        pltpu.VMEM((BPW, FACTOR), jnp.float32),
        pltpu.VMEM((BPW, FACTOR), jnp.float32),
        pltpu.VMEM((FACTOR,), jnp.float32),
        pltpu.VMEM((L,), jnp.float32),
        pltpu.VMEM((BPW,), jnp.float32),
        pltpu.SemaphoreType.DMA,
        pltpu.SemaphoreType.DMA,
    ],
)
def _gmf_compute(u_hbm, i_hbm, w_hbm, b_hbm, out_hbm,
                 u_rows, i_rows, w_v, b_v, out_v, sem_u, sem_i):
    wid = lax.axis_index("s") * NC + lax.axis_index("c")
    base = wid * BPW
    pltpu.sync_copy(w_hbm, w_v)
    pltpu.sync_copy(b_hbm, b_v)
    w = [w_v[pl.ds(c * L, L)] for c in range(NCHUNK)]
    bv = b_v[...]
    iota = lax.iota(jnp.int32, L)

    cu = pltpu.async_copy(
        u_hbm.at[pl.ds(base, BPW), pl.ds(0, FACTOR)], u_rows, sem_u)
    ci = pltpu.async_copy(
        i_hbm.at[pl.ds(base, BPW), pl.ds(0, FACTOR)], i_rows, sem_i)
    cu.wait()
    ci.wait()

    def group(gi, carry):
        rowbase = gi * L
        logit = jnp.zeros((L,), jnp.float32)
        for e in range(L):
            row = rowbase + e
            acc = None
            for c in range(NCHUNK):
                u = u_rows[row, pl.ds(c * L, L)]
                iv = i_rows[row, pl.ds(c * L, L)]
                p = u * iv * w[c]
                acc = p if acc is None else acc + p
            logit = jnp.where(iota == e, jnp.sum(acc), logit)
        logit = logit + bv
        out_v[pl.ds(rowbase, L)] = 1.0 / (1.0 + jnp.exp(-logit))
        return carry

    lax.fori_loop(0, BPW // L, group, 0)

    pltpu.sync_copy(out_v, out_hbm.at[pl.ds(base, BPW)])


def kernel(user, item, user_table, item_table, W, b):
    u_buf, i_buf = _gmf_scan(user, item, user_table.T, item_table.T)
    w64 = W.reshape((FACTOR,))
    b16 = jnp.broadcast_to(b, (L,))
    out = _gmf_compute(u_buf, i_buf, w64, b16)
    return out.reshape((BATCH, 1))
